# baseline (speedup 1.0000x reference)
import functools

import jax
import jax.numpy as jnp
from jax import lax
from jax.experimental import pallas as pl
from jax.experimental.pallas import tpu as pltpu
from jax.experimental.pallas import tpu_sc as plsc

N = 10000
E = 160000
D_IN = 256
D_HID = 256
D_OUT = 64

NC = 2
NS = 16
LANES = 16

BN = 2000
NB = N // BN

_f32 = jnp.float32
_i32 = jnp.int32

_PE = 163840
_PCH = _PE // 128
_TPR = _PCH // (NC * NS)
_CPT = _PCH // NS
_ACC1D = 10240
NP = 10240


def _sc_mesh():
    return plsc.VectorSubcoreMesh(
        core_axis_name="c", subcore_axis_name="s", num_cores=NC, num_subcores=NS
    )


def _chunks(total, step):
    out = []
    r = 0
    while r < total:
        out.append((r, min(step, total - r)))
        r += step
    return out


@functools.partial(
    pl.kernel,
    out_type=jax.ShapeDtypeStruct((NC * _ACC1D,), _f32),
    mesh=_sc_mesh(),
    scratch_types=[
        pltpu.VMEM((640,), _f32),
        pltpu.VMEM((128,), _f32),
        pltpu.VMEM((_TPR, 128), _i32),
        pltpu.VMEM_SHARED((_ACC1D,), _f32),
    ],
)
def _deg_kernel(dst_hbm, out_hbm, zbuf, ones_v, didx, acc):
    c = lax.axis_index("c")
    s = lax.axis_index("s")
    for k in range(640 // LANES):
        zbuf[pl.ds(k * LANES, LANES)] = jnp.zeros((LANES,), _f32)
    for k in range(128 // LANES):
        ones_v[pl.ds(k * LANES, LANES)] = jnp.ones((LANES,), _f32)
    tid = c * NS + s
    pltpu.sync_copy(dst_hbm.at[pl.ds(pl.multiple_of(tid * _TPR, 8), _TPR)],
                    didx)
    pltpu.sync_copy(zbuf, acc.at[pl.ds(s * 640, 640)])
    plsc.subcore_barrier()

    def body(j, carry):
        pltpu.sync_copy(ones_v, acc.at[didx.at[j]], add=True)
        return carry

    lax.fori_loop(0, _TPR, body, 0)
    plsc.subcore_barrier()
    pltpu.sync_copy(acc.at[pl.ds(s * 640, 640)], zbuf)
    pltpu.sync_copy(zbuf,
                    out_hbm.at[pl.ds(pl.multiple_of(c * _ACC1D + s * 640, 8),
                                     640)])


def _make_agg(Dh):
    ept = E // NS
    nch = ept // 128
    tail = ept - nch * 128
    rpt = NP // NS

    @functools.partial(
        pl.kernel,
        out_type=jax.ShapeDtypeStruct((2 * NP, Dh), _f32),
        mesh=_sc_mesh(),
        scratch_types=[
            pltpu.VMEM((128,), _i32),
            pltpu.VMEM((128,), _i32),
            pltpu.VMEM((tail,), _i32),
            pltpu.VMEM((tail,), _i32),
            pltpu.VMEM((128, Dh), _f32),
            pltpu.VMEM((tail, Dh), _f32),
            pltpu.VMEM_SHARED((NP, Dh), _f32),
            pltpu.SemaphoreType.DMA,
        ],
    )
    def agg(g_hbm, src_hbm, dst_hbm, zero_hbm, out_hbm,
            src_v, dst_v, srct_v, dstt_v, rows_v, rowst_v, acc, sem):
        c = lax.axis_index("c")
        s = lax.axis_index("s")
        cN = c * N
        r0 = s * rpt
        pltpu.sync_copy(zero_hbm, rows_v)
        for r, cw in _chunks(rpt, 128):
            pltpu.sync_copy(rows_v.at[pl.ds(0, cw)], acc.at[pl.ds(r0 + r, cw)])
        plsc.subcore_barrier()

        base = s * ept

        def body(j, carry):
            off = pl.multiple_of(base + j * 128, 8)
            pltpu.sync_copy(src_hbm.at[pl.ds(off, 128)], src_v)
            pltpu.sync_copy(dst_hbm.at[pl.ds(off, 128)], dst_v)
            for k in range(128 // LANES):
                sl = pl.ds(k * LANES, LANES)
                src_v[sl] = src_v[sl] + cN
            pltpu.async_copy(g_hbm.at[src_v], rows_v, sem).wait()
            pltpu.sync_copy(rows_v, acc.at[dst_v], add=True)
            return carry

        lax.fori_loop(0, nch, body, 0)
        offt = pl.multiple_of(base + nch * 128, 8)
        pltpu.sync_copy(src_hbm.at[pl.ds(offt, tail)], srct_v)
        pltpu.sync_copy(dst_hbm.at[pl.ds(offt, tail)], dstt_v)
        srct_v[...] = srct_v[...] + cN
        pltpu.async_copy(g_hbm.at[srct_v], rowst_v, sem).wait()
        pltpu.sync_copy(rowst_v, acc.at[dstt_v], add=True)
        plsc.subcore_barrier()
        cNP = c * NP
        for r, cw in _chunks(rpt, 128):
            pltpu.sync_copy(acc.at[pl.ds(r0 + r, cw)], rows_v.at[pl.ds(0, cw)])
            pltpu.sync_copy(rows_v.at[pl.ds(0, cw)],
                            out_hbm.at[pl.ds(pl.multiple_of(cNP + r0 + r, 8),
                                             cw)])

    return agg


_agg128 = _make_agg(D_HID // 2)


def _dis(hist_blk):
    deg = hist_blk[:, 0:1] + hist_blk[:, 1:2] + 1.0
    return lax.rsqrt(deg)


def _tc1_body(hist_ref, x_ref, w_ref, out_ref):
    dis = _dis(hist_ref[...])
    h = jnp.dot(x_ref[...], w_ref[...], precision=lax.Precision.HIGHEST,
                preferred_element_type=_f32)
    out_ref[0] = dis * h


_tc1 = pl.pallas_call(
    _tc1_body,
    grid=(NC, NB),
    in_specs=[
        pl.BlockSpec((BN, 2), lambda c, i: (i, 0)),
        pl.BlockSpec((BN, D_IN), lambda c, i: (i, 0)),
        pl.BlockSpec((D_IN, D_HID // 2), lambda c, i: (0, c)),
    ],
    out_specs=pl.BlockSpec((1, BN, D_HID // 2), lambda c, i: (c, i, 0)),
    out_shape=jax.ShapeDtypeStruct((NC, N, D_HID // 2), _f32),
)


def _tc2_body(hist_ref, acc_ref, g_ref, b1_ref, out_ref):
    dis = _dis(hist_ref[...])
    h = jnp.concatenate([acc_ref[0] + g_ref[0], acc_ref[1] + g_ref[1]], axis=1)
    hidden = dis * h + b1_ref[...]
    q = dis * jnp.maximum(hidden, 0.0)
    out_ref[0] = q[:, : D_HID // 2]
    out_ref[1] = q[:, D_HID // 2:]


_tc2 = pl.pallas_call(
    _tc2_body,
    grid=(NB,),
    in_specs=[
        pl.BlockSpec((BN, 2), lambda i: (i, 0)),
        pl.BlockSpec((NC, BN, D_HID // 2), lambda i: (0, i, 0)),
        pl.BlockSpec((NC, BN, D_HID // 2), lambda i: (0, i, 0)),
        pl.BlockSpec((1, D_HID), lambda i: (0, 0)),
    ],
    out_specs=pl.BlockSpec((NC, BN, D_HID // 2), lambda i: (0, i, 0)),
    out_shape=jax.ShapeDtypeStruct((NC, N, D_HID // 2), _f32),
)


def _tc3_body(hist_ref, acc_ref, q_ref, w2_ref, b2_ref, logsm_ref, out_ref):
    dis = _dis(hist_ref[...])
    m = jnp.concatenate([acc_ref[0] + q_ref[0], acc_ref[1] + q_ref[1]], axis=1)
    o = dis * jnp.dot(m, w2_ref[...], precision=lax.Precision.HIGHEST,
                      preferred_element_type=_f32) + b2_ref[...]
    out_ref[...] = o
    mx = jnp.max(o, axis=1, keepdims=True)
    lse = jnp.log(jnp.sum(jnp.exp(o - mx), axis=1, keepdims=True)) + mx
    logsm_ref[...] = o - lse


_tc3 = pl.pallas_call(
    _tc3_body,
    grid=(NB,),
    in_specs=[
        pl.BlockSpec((BN, 2), lambda i: (i, 0)),
        pl.BlockSpec((NC, BN, D_HID // 2), lambda i: (0, i, 0)),
        pl.BlockSpec((NC, BN, D_HID // 2), lambda i: (0, i, 0)),
        pl.BlockSpec((D_HID, D_OUT), lambda i: (0, 0)),
        pl.BlockSpec((1, D_OUT), lambda i: (0, 0)),
    ],
    out_specs=[
        pl.BlockSpec((BN, D_OUT), lambda i: (i, 0)),
        pl.BlockSpec((BN, D_OUT), lambda i: (i, 0)),
    ],
    out_shape=[
        jax.ShapeDtypeStruct((N, D_OUT), _f32),
        jax.ShapeDtypeStruct((N, D_OUT), _f32),
    ],
)


def kernel(x, edge_index, W1, b1, W2, b2):
    edge_index = edge_index.astype(_i32)
    src = jnp.sort(edge_index[0])
    dst = edge_index[1]

    pad = _PE - E
    srcp = jnp.concatenate([src, jnp.zeros((pad,), _i32)])
    padbin = N + (jnp.arange(pad, dtype=_i32) % (NP - N))
    dstp = jnp.concatenate([dst, padbin])
    dst2d = dstp.reshape(_PCH, 128)
    z128 = jnp.zeros((128, D_HID // 2), _f32)

    hist = _deg_kernel(dst2d).reshape(NC, _ACC1D)[:, :N]
    histT = hist.T

    g1 = _tc1(histT, x, W1)
    acc1 = _agg128(g1.reshape(2 * N, D_HID // 2), src, dst, z128)
    q = _tc2(histT, acc1.reshape(NC, NP, D_HID // 2), g1,
             b1.reshape(1, D_HID))
    acc2 = _agg128(q.reshape(2 * N, D_HID // 2), src, dst, z128)
    logsm, out = _tc3(histT, acc2.reshape(NC, NP, D_HID // 2), q,
                      W2, b2.reshape(1, D_OUT))
    return (logsm, out)


# speedup vs baseline: 1.7321x; 1.7321x over previous
import functools

import jax
import jax.numpy as jnp
from jax import lax
from jax.experimental import pallas as pl
from jax.experimental.pallas import tpu as pltpu
from jax.experimental.pallas import tpu_sc as plsc

N = 10000
E = 160000
D_IN = 256
D_HID = 256
D_OUT = 64

NC = 2
NS = 16
LANES = 16

BN = 2000
NB = N // BN

_f32 = jnp.float32
_i32 = jnp.int32

_PE = 163840
_PCH = _PE // 128
_TPR = _PCH // (NC * NS)
_CPT = _PCH // NS
_ACC1D = 10240
NP = 10240


def _sc_mesh():
    return plsc.VectorSubcoreMesh(
        core_axis_name="c", subcore_axis_name="s", num_cores=NC, num_subcores=NS
    )


def _chunks(total, step):
    out = []
    r = 0
    while r < total:
        out.append((r, min(step, total - r)))
        r += step
    return out


@functools.partial(
    pl.kernel,
    out_type=jax.ShapeDtypeStruct((NC * _ACC1D,), _f32),
    mesh=_sc_mesh(),
    scratch_types=[
        pltpu.VMEM((640,), _f32),
        pltpu.VMEM((128,), _f32),
        pltpu.VMEM((_TPR, 128), _i32),
        pltpu.VMEM_SHARED((_ACC1D,), _f32),
    ],
)
def _deg_kernel(dst_hbm, out_hbm, zbuf, ones_v, didx, acc):
    c = lax.axis_index("c")
    s = lax.axis_index("s")
    for k in range(640 // LANES):
        zbuf[pl.ds(k * LANES, LANES)] = jnp.zeros((LANES,), _f32)
    for k in range(128 // LANES):
        ones_v[pl.ds(k * LANES, LANES)] = jnp.ones((LANES,), _f32)
    tid = c * NS + s
    pltpu.sync_copy(dst_hbm.at[pl.ds(pl.multiple_of(tid * _TPR, 8), _TPR)],
                    didx)
    pltpu.sync_copy(zbuf, acc.at[pl.ds(s * 640, 640)])
    plsc.subcore_barrier()

    def body(j, carry):
        pltpu.sync_copy(ones_v, acc.at[didx.at[j]], add=True)
        return carry

    lax.fori_loop(0, _TPR, body, 0)
    plsc.subcore_barrier()
    pltpu.sync_copy(acc.at[pl.ds(s * 640, 640)], zbuf)
    pltpu.sync_copy(zbuf,
                    out_hbm.at[pl.ds(pl.multiple_of(c * _ACC1D + s * 640, 8),
                                     640)])


def _make_agg(Dh):
    ept = E // NS
    nch = ept // 128
    tail = ept - nch * 128
    rpt = NP // NS

    @functools.partial(
        pl.kernel,
        out_type=jax.ShapeDtypeStruct((2 * NP, Dh), _f32),
        mesh=_sc_mesh(),
        scratch_types=[
            pltpu.VMEM((128,), _i32),
            pltpu.VMEM((128,), _i32),
            pltpu.VMEM((tail,), _i32),
            pltpu.VMEM((tail,), _i32),
            pltpu.VMEM((128, Dh), _f32),
            pltpu.VMEM((tail, Dh), _f32),
            pltpu.VMEM_SHARED((NP, Dh), _f32),
            pltpu.SemaphoreType.DMA,
        ],
    )
    def agg(g_hbm, src_hbm, dst_hbm, zero_hbm, out_hbm,
            src_v, dst_v, srct_v, dstt_v, rows_v, rowst_v, acc, sem):
        c = lax.axis_index("c")
        s = lax.axis_index("s")
        cN = c * N
        r0 = s * rpt
        pltpu.sync_copy(zero_hbm, rows_v)
        for r, cw in _chunks(rpt, 128):
            pltpu.sync_copy(rows_v.at[pl.ds(0, cw)], acc.at[pl.ds(r0 + r, cw)])
        plsc.subcore_barrier()

        base = s * ept

        def body(j, carry):
            off = pl.multiple_of(base + j * 128, 8)
            pltpu.sync_copy(src_hbm.at[pl.ds(off, 128)], src_v)
            pltpu.sync_copy(dst_hbm.at[pl.ds(off, 128)], dst_v)
            for k in range(128 // LANES):
                sl = pl.ds(k * LANES, LANES)
                src_v[sl] = src_v[sl] + cN
            pltpu.async_copy(g_hbm.at[src_v], rows_v, sem).wait()
            pltpu.sync_copy(rows_v, acc.at[dst_v], add=True)
            return carry

        lax.fori_loop(0, nch, body, 0)
        offt = pl.multiple_of(base + nch * 128, 8)
        pltpu.sync_copy(src_hbm.at[pl.ds(offt, tail)], srct_v)
        pltpu.sync_copy(dst_hbm.at[pl.ds(offt, tail)], dstt_v)
        srct_v[...] = srct_v[...] + cN
        pltpu.async_copy(g_hbm.at[srct_v], rowst_v, sem).wait()
        pltpu.sync_copy(rowst_v, acc.at[dstt_v], add=True)
        plsc.subcore_barrier()
        cNP = c * NP
        for r, cw in _chunks(rpt, 128):
            pltpu.sync_copy(acc.at[pl.ds(r0 + r, cw)], rows_v.at[pl.ds(0, cw)])
            pltpu.sync_copy(rows_v.at[pl.ds(0, cw)],
                            out_hbm.at[pl.ds(pl.multiple_of(cNP + r0 + r, 8),
                                             cw)])

    return agg


_agg128 = _make_agg(D_HID // 2)


def _dis(hist_blk):
    deg = hist_blk[:, 0:1] + hist_blk[:, 1:2] + 1.0
    return lax.rsqrt(deg)


def _tc1_body(hist_ref, x_ref, w_ref, out_ref):
    dis = _dis(hist_ref[...])
    h = dis * jnp.dot(x_ref[...], w_ref[...], precision=lax.Precision.HIGHEST,
                      preferred_element_type=_f32)
    out_ref[0] = h[:, : D_HID // 2]
    out_ref[1] = h[:, D_HID // 2:]


_tc1 = pl.pallas_call(
    _tc1_body,
    grid=(NB,),
    in_specs=[
        pl.BlockSpec((BN, 2), lambda i: (i, 0)),
        pl.BlockSpec((BN, D_IN), lambda i: (i, 0)),
        pl.BlockSpec((D_IN, D_HID), lambda i: (0, 0)),
    ],
    out_specs=pl.BlockSpec((NC, BN, D_HID // 2), lambda i: (0, i, 0)),
    out_shape=jax.ShapeDtypeStruct((NC, N, D_HID // 2), _f32),
)


def _tc2_body(hist_ref, acc_ref, g_ref, b1_ref, out_ref):
    dis = _dis(hist_ref[...])
    h = jnp.concatenate([acc_ref[0] + g_ref[0], acc_ref[1] + g_ref[1]], axis=1)
    hidden = dis * h + b1_ref[...]
    q = dis * jnp.maximum(hidden, 0.0)
    out_ref[0] = q[:, : D_HID // 2]
    out_ref[1] = q[:, D_HID // 2:]


_tc2 = pl.pallas_call(
    _tc2_body,
    grid=(NB,),
    in_specs=[
        pl.BlockSpec((BN, 2), lambda i: (i, 0)),
        pl.BlockSpec((NC, BN, D_HID // 2), lambda i: (0, i, 0)),
        pl.BlockSpec((NC, BN, D_HID // 2), lambda i: (0, i, 0)),
        pl.BlockSpec((1, D_HID), lambda i: (0, 0)),
    ],
    out_specs=pl.BlockSpec((NC, BN, D_HID // 2), lambda i: (0, i, 0)),
    out_shape=jax.ShapeDtypeStruct((NC, N, D_HID // 2), _f32),
)


def _tc3_body(hist_ref, acc_ref, q_ref, w2_ref, b2_ref, logsm_ref, out_ref):
    dis = _dis(hist_ref[...])
    m = jnp.concatenate([acc_ref[0] + q_ref[0], acc_ref[1] + q_ref[1]], axis=1)
    o = dis * jnp.dot(m, w2_ref[...], precision=lax.Precision.HIGHEST,
                      preferred_element_type=_f32) + b2_ref[...]
    out_ref[...] = o
    mx = jnp.max(o, axis=1, keepdims=True)
    lse = jnp.log(jnp.sum(jnp.exp(o - mx), axis=1, keepdims=True)) + mx
    logsm_ref[...] = o - lse


_tc3 = pl.pallas_call(
    _tc3_body,
    grid=(NB,),
    in_specs=[
        pl.BlockSpec((BN, 2), lambda i: (i, 0)),
        pl.BlockSpec((NC, BN, D_HID // 2), lambda i: (0, i, 0)),
        pl.BlockSpec((NC, BN, D_HID // 2), lambda i: (0, i, 0)),
        pl.BlockSpec((D_HID, D_OUT), lambda i: (0, 0)),
        pl.BlockSpec((1, D_OUT), lambda i: (0, 0)),
    ],
    out_specs=[
        pl.BlockSpec((BN, D_OUT), lambda i: (i, 0)),
        pl.BlockSpec((BN, D_OUT), lambda i: (i, 0)),
    ],
    out_shape=[
        jax.ShapeDtypeStruct((N, D_OUT), _f32),
        jax.ShapeDtypeStruct((N, D_OUT), _f32),
    ],
)


def kernel(x, edge_index, W1, b1, W2, b2):
    edge_index = edge_index.astype(_i32)
    src = edge_index[0]
    dst = edge_index[1]

    pad = _PE - E
    srcp = jnp.concatenate([src, jnp.zeros((pad,), _i32)])
    padbin = N + (jnp.arange(pad, dtype=_i32) % (NP - N))
    dstp = jnp.concatenate([dst, padbin])
    dst2d = dstp.reshape(_PCH, 128)
    z128 = jnp.zeros((128, D_HID // 2), _f32)

    hist = _deg_kernel(dst2d).reshape(NC, _ACC1D)[:, :N]
    histT = hist.T

    g1 = _tc1(histT, x, W1)
    acc1 = _agg128(g1.reshape(2 * N, D_HID // 2), src, dst, z128)
    q = _tc2(histT, acc1.reshape(NC, NP, D_HID // 2), g1,
             b1.reshape(1, D_HID))
    acc2 = _agg128(q.reshape(2 * N, D_HID // 2), src, dst, z128)
    logsm, out = _tc3(histT, acc2.reshape(NC, NP, D_HID // 2), q,
                      W2, b2.reshape(1, D_OUT))
    return (logsm, out)
